# baseline (speedup 1.0000x reference)
import functools

import jax
import jax.numpy as jnp
from jax import lax
from jax.experimental import pallas as pl
from jax.experimental.pallas import tpu as pltpu
from jax.experimental.pallas import tpu_sc as plsc

VOCAB = 100000
EMBED_DIM = 128
BATCH = 16384
HIST_LEN = 50
OUT_DIM = 2

PADW = 16
VBLK = 16384
VGRID = -(-VOCAB // VBLK)

NC, NS = 2, 16
NW = NC * NS
BPW = BATCH // NW
CHUNK = 64
GRPC = CHUNK // 16


def _rtne_bf16_hi(u):
    return (u + 0x7FFF + ((u >> 16) & 1)) >> 16


def _proj_body(emb_ref, w_ref, b_ref, out_ref):
    pT = (
        lax.dot_general(
            w_ref[...],
            emb_ref[...],
            (((1,), (1,)), ((), ())),
            preferred_element_type=jnp.float32,
        )
        + b_ref[...]
    )
    u = lax.bitcast_convert_type(pT, jnp.uint32)
    r = _rtne_bf16_hi(u)
    word = r[0:1, :] | (r[1:2, :] << 16)
    out_ref[...] = lax.bitcast_convert_type(word[0], jnp.int32)


def _project_table(emb_w, w_pad, b_pad):
    return pl.pallas_call(
        _proj_body,
        grid=(VGRID,),
        in_specs=[
            pl.BlockSpec((VBLK, EMBED_DIM), lambda i: (i, 0)),
            pl.BlockSpec((PADW, EMBED_DIM), lambda i: (0, 0)),
            pl.BlockSpec((PADW, 1), lambda i: (0, 0)),
        ],
        out_specs=pl.BlockSpec((VBLK,), lambda i: (i,)),
        out_shape=jax.ShapeDtypeStruct((VOCAB,), jnp.int32),
    )(emb_w, w_pad, b_pad)


TBLK = 512


def _tr_body(i_ref, t_ref, out_ref):
    out_ref[...] = lax.dot_general(
        i_ref[...],
        t_ref[...],
        (((1,), (1,)), ((), ())),
        preferred_element_type=jnp.float32,
    )


def _untranspose(out_t, ident):
    return pl.pallas_call(
        _tr_body,
        grid=(BATCH // TBLK,),
        in_specs=[
            pl.BlockSpec((TBLK, TBLK), lambda i: (0, 0)),
            pl.BlockSpec((OUT_DIM, TBLK), lambda i: (0, i)),
        ],
        out_specs=pl.BlockSpec((TBLK, OUT_DIM), lambda i: (i, 0)),
        out_shape=jax.ShapeDtypeStruct((BATCH, OUT_DIM), jnp.float32),
    )(ident, out_t)


@functools.partial(
    pl.kernel,
    out_type=jax.ShapeDtypeStruct((OUT_DIM, BATCH), jnp.float32),
    mesh=plsc.VectorSubcoreMesh(core_axis_name="c", subcore_axis_name="s"),
    scratch_types=[
        pltpu.VMEM((VOCAB,), jnp.int32),
        pltpu.VMEM((CHUNK, HIST_LEN), jnp.int32),
        pltpu.VMEM((CHUNK, HIST_LEN), jnp.int32),
        pltpu.VMEM((1, BPW), jnp.float32),
        pltpu.VMEM((1, BPW), jnp.float32),
        pltpu.SemaphoreType.DMA,
        pltpu.SemaphoreType.DMA,
        pltpu.SemaphoreType.DMA,
    ],
    compiler_params=pltpu.CompilerParams(
        use_tc_tiling_on_sc=True, needs_layout_passes=False
    ),
)
def _sc_pool(
    proj_hbm, s_hbm, out_hbm, table_v, s_va, s_vb, out0_v, out1_v, sem_t, sem_a, sem_b
):
    wid = lax.axis_index("s") * NC + lax.axis_index("c")
    base = wid * BPW
    nchunk = BPW // CHUNK
    bufs = (s_va, s_vb)
    sems = (sem_a, sem_b)
    cp_t = pltpu.async_copy(proj_hbm, table_v, sem_t)
    pending = [
        pltpu.async_copy(s_hbm.at[pl.ds(base, CHUNK), :], s_va, sem_a),
        None,
    ]
    cp_t.wait()

    lane = lax.iota(jnp.int32, 16)
    zeros_i = jnp.zeros((16,), jnp.int32)
    zero = jnp.zeros((16,), jnp.float32)
    mask_hi = jnp.full((16,), jnp.int32(-65536))

    for c in range(nchunk):
        if c + 1 < nchunk:
            pending[(c + 1) % 2] = pltpu.async_copy(
                s_hbm.at[pl.ds(base + (c + 1) * CHUNK, CHUNK), :],
                bufs[(c + 1) % 2],
                sems[(c + 1) % 2],
            )
        pending[c % 2].wait()
        s_v = bufs[c % 2]

        def g_body(g, carry):
            row = g * 16 + lane
            acc0, acc1 = zero, zero
            for l in range(HIST_LEN):
                v = plsc.load_gather(s_v, [row, jnp.full((16,), l, jnp.int32)])
                w = plsc.load_gather(table_v, [v])
                acc0 = acc0 + plsc.bitcast(w << 16, jnp.float32)
                acc1 = acc1 + plsc.bitcast(w & mask_hi, jnp.float32)
            pos = c * CHUNK + row
            plsc.store_scatter(out0_v, [zeros_i, pos], acc0)
            plsc.store_scatter(out1_v, [zeros_i, pos], acc1)
            return carry

        lax.fori_loop(0, GRPC, g_body, 0)

    pltpu.sync_copy(out0_v, out_hbm.at[0:1, pl.ds(base, BPW)])
    pltpu.sync_copy(out1_v, out_hbm.at[1:2, pl.ds(base, BPW)])


def kernel(s, emb_w, lin_w, lin_b):
    s32 = s.astype(jnp.int32)
    w_pad = jnp.zeros((PADW, EMBED_DIM), jnp.float32).at[:OUT_DIM, :].set(lin_w)
    b_pad = jnp.zeros((PADW, 1), jnp.float32).at[:OUT_DIM, 0].set(lin_b)
    proj = _project_table(emb_w, w_pad, b_pad)
    out_t = _sc_pool(proj, s32)
    ident = jnp.eye(TBLK, dtype=jnp.float32)
    return _untranspose(out_t, ident)


# speedup vs baseline: 1.1970x; 1.1970x over previous
import functools

import jax
import jax.numpy as jnp
from jax import lax
from jax.experimental import pallas as pl
from jax.experimental.pallas import tpu as pltpu
from jax.experimental.pallas import tpu_sc as plsc

VOCAB = 100000
EMBED_DIM = 128
BATCH = 16384
HIST_LEN = 50
OUT_DIM = 2

PADW = 16
VBLK = 16384
VGRID = -(-VOCAB // VBLK)

NC, NS = 2, 16
NW = NC * NS
BPW = BATCH // NW
CHUNK = 64
GRPC = CHUNK // 16


def _rtne_bf16_hi(u):
    return (u + 0x7FFF + ((u >> 16) & 1)) >> 16


def _proj_body(emb_ref, w_ref, b_ref, out_ref):
    pT = (
        lax.dot_general(
            w_ref[...],
            emb_ref[...],
            (((1,), (1,)), ((), ())),
            preferred_element_type=jnp.float32,
        )
        + b_ref[...]
    )
    u = lax.bitcast_convert_type(pT, jnp.uint32)
    r = _rtne_bf16_hi(u)
    word = r[0:1, :] | (r[1:2, :] << 16)
    out_ref[...] = lax.bitcast_convert_type(word[0], jnp.int32)


def _project_table(emb_w, w_pad, b_pad):
    return pl.pallas_call(
        _proj_body,
        grid=(VGRID,),
        in_specs=[
            pl.BlockSpec((VBLK, EMBED_DIM), lambda i: (i, 0)),
            pl.BlockSpec((PADW, EMBED_DIM), lambda i: (0, 0)),
            pl.BlockSpec((PADW, 1), lambda i: (0, 0)),
        ],
        out_specs=pl.BlockSpec((VBLK,), lambda i: (i,)),
        out_shape=jax.ShapeDtypeStruct((VOCAB,), jnp.int32),
    )(emb_w, w_pad, b_pad)


@functools.partial(
    pl.kernel,
    out_type=jax.ShapeDtypeStruct((BATCH, OUT_DIM), jnp.float32),
    mesh=plsc.VectorSubcoreMesh(core_axis_name="c", subcore_axis_name="s"),
    scratch_types=[
        pltpu.VMEM((VOCAB,), jnp.int32),
        pltpu.VMEM((CHUNK, HIST_LEN), jnp.int32),
        pltpu.VMEM((CHUNK, HIST_LEN), jnp.int32),
        pltpu.VMEM((CHUNK, OUT_DIM), jnp.float32),
        pltpu.SemaphoreType.DMA,
        pltpu.SemaphoreType.DMA,
        pltpu.SemaphoreType.DMA,
    ],
    compiler_params=pltpu.CompilerParams(
        use_tc_tiling_on_sc=True, needs_layout_passes=False
    ),
)
def _sc_pool(proj_hbm, s_hbm, out_hbm, table_v, s_va, s_vb, out_v, sem_t, sem_a, sem_b):
    wid = lax.axis_index("s") * NC + lax.axis_index("c")
    base = wid * BPW
    nchunk = BPW // CHUNK
    bufs = (s_va, s_vb)
    sems = (sem_a, sem_b)
    cp_t = pltpu.async_copy(proj_hbm, table_v, sem_t)
    pending = [
        pltpu.async_copy(s_hbm.at[pl.ds(base, CHUNK), :], s_va, sem_a),
        None,
    ]
    cp_t.wait()

    lane = lax.iota(jnp.int32, 16)
    zeros_i = jnp.zeros((16,), jnp.int32)
    zero = jnp.zeros((16,), jnp.float32)
    mask_hi = jnp.full((16,), jnp.int32(-65536))

    for c in range(nchunk):
        if c + 1 < nchunk:
            pending[(c + 1) % 2] = pltpu.async_copy(
                s_hbm.at[pl.ds(base + (c + 1) * CHUNK, CHUNK), :],
                bufs[(c + 1) % 2],
                sems[(c + 1) % 2],
            )
        pending[c % 2].wait()
        s_v = bufs[c % 2]

        def g_body(g, carry):
            row = g * 16 + lane
            acc0, acc1 = zero, zero
            for l in range(HIST_LEN):
                v = plsc.load_gather(s_v, [row, jnp.full((16,), l, jnp.int32)])
                w = plsc.load_gather(table_v, [v])
                acc0 = acc0 + plsc.bitcast(w << 16, jnp.float32)
                acc1 = acc1 + plsc.bitcast(w & mask_hi, jnp.float32)
            plsc.store_scatter(out_v, [row, zeros_i], acc0)
            plsc.store_scatter(out_v, [row, zeros_i + 1], acc1)
            return carry

        lax.fori_loop(0, GRPC, g_body, 0)
        pltpu.sync_copy(out_v, out_hbm.at[pl.ds(base + c * CHUNK, CHUNK), :])


def kernel(s, emb_w, lin_w, lin_b):
    s32 = s.astype(jnp.int32)
    w_pad = jnp.zeros((PADW, EMBED_DIM), jnp.float32).at[:OUT_DIM, :].set(lin_w)
    b_pad = jnp.zeros((PADW, 1), jnp.float32).at[:OUT_DIM, 0].set(lin_b)
    proj = _project_table(emb_w, w_pad, b_pad)
    return _sc_pool(proj, s32)


# speedup vs baseline: 1.2165x; 1.0163x over previous
import functools

import jax
import jax.numpy as jnp
from jax import lax
from jax.experimental import pallas as pl
from jax.experimental.pallas import tpu as pltpu
from jax.experimental.pallas import tpu_sc as plsc

VOCAB = 100000
EMBED_DIM = 128
BATCH = 16384
HIST_LEN = 50
OUT_DIM = 2

PADW = 16
VBLK = 16384
VGRID = -(-VOCAB // VBLK)

NC, NS = 2, 16
NW = NC * NS
BPW = BATCH // NW
CHUNK = 64
GRPC = CHUNK // 16


def _rtne_bf16_hi(u):
    return (u + 0x7FFF + ((u >> 16) & 1)) >> 16


def _proj_body(emb_ref, w_ref, b_ref, out_ref):
    pT = (
        lax.dot_general(
            w_ref[...],
            emb_ref[...],
            (((1,), (1,)), ((), ())),
            preferred_element_type=jnp.float32,
        )
        + b_ref[...]
    )
    u = lax.bitcast_convert_type(pT, jnp.uint32)
    r = _rtne_bf16_hi(u)
    word = r[0:1, :] | (r[1:2, :] << 16)
    out_ref[...] = lax.bitcast_convert_type(word[0], jnp.int32)


def _project_table(emb_w, w_pad, b_pad):
    return pl.pallas_call(
        _proj_body,
        grid=(VGRID,),
        in_specs=[
            pl.BlockSpec((VBLK, EMBED_DIM), lambda i: (i, 0)),
            pl.BlockSpec((OUT_DIM, EMBED_DIM), lambda i: (0, 0)),
            pl.BlockSpec((OUT_DIM, 1), lambda i: (0, 0)),
        ],
        out_specs=pl.BlockSpec((VBLK,), lambda i: (i,)),
        out_shape=jax.ShapeDtypeStruct((VOCAB,), jnp.int32),
    )(emb_w, w_pad, b_pad)


@functools.partial(
    pl.kernel,
    out_type=jax.ShapeDtypeStruct((BATCH, OUT_DIM), jnp.float32),
    mesh=plsc.VectorSubcoreMesh(core_axis_name="c", subcore_axis_name="s"),
    scratch_types=[
        pltpu.VMEM((VOCAB,), jnp.int32),
        pltpu.VMEM((CHUNK, HIST_LEN), jnp.int32),
        pltpu.VMEM((CHUNK, HIST_LEN), jnp.int32),
        pltpu.VMEM((CHUNK, OUT_DIM), jnp.float32),
        pltpu.SemaphoreType.DMA,
        pltpu.SemaphoreType.DMA,
        pltpu.SemaphoreType.DMA,
    ],
    compiler_params=pltpu.CompilerParams(
        use_tc_tiling_on_sc=True, needs_layout_passes=False
    ),
)
def _sc_pool(proj_hbm, s_hbm, out_hbm, table_v, s_va, s_vb, out_v, sem_t, sem_a, sem_b):
    wid = lax.axis_index("s") * NC + lax.axis_index("c")
    base = wid * BPW
    nchunk = BPW // CHUNK
    bufs = (s_va, s_vb)
    sems = (sem_a, sem_b)
    cp_t = pltpu.async_copy(proj_hbm, table_v, sem_t)
    pending = [
        pltpu.async_copy(s_hbm.at[pl.ds(base, CHUNK), :], s_va, sem_a),
        None,
    ]
    cp_t.wait()

    lane = lax.iota(jnp.int32, 16)
    zeros_i = jnp.zeros((16,), jnp.int32)
    zero = jnp.zeros((16,), jnp.float32)
    mask_hi = jnp.full((16,), jnp.int32(-65536))

    for c in range(nchunk):
        if c + 1 < nchunk:
            pending[(c + 1) % 2] = pltpu.async_copy(
                s_hbm.at[pl.ds(base + (c + 1) * CHUNK, CHUNK), :],
                bufs[(c + 1) % 2],
                sems[(c + 1) % 2],
            )
        pending[c % 2].wait()
        s_v = bufs[c % 2]

        def g_body(g, carry):
            row = g * 16 + lane
            acc0, acc1 = zero, zero
            for l in range(HIST_LEN):
                v = plsc.load_gather(s_v, [row, jnp.full((16,), l, jnp.int32)])
                w = plsc.load_gather(table_v, [v])
                acc0 = acc0 + plsc.bitcast(w << 16, jnp.float32)
                acc1 = acc1 + plsc.bitcast(w & mask_hi, jnp.float32)
            plsc.store_scatter(out_v, [row, zeros_i], acc0)
            plsc.store_scatter(out_v, [row, zeros_i + 1], acc1)
            return carry

        lax.fori_loop(0, GRPC, g_body, 0)
        pltpu.sync_copy(out_v, out_hbm.at[pl.ds(base + c * CHUNK, CHUNK), :])


def kernel(s, emb_w, lin_w, lin_b):
    s32 = s.astype(jnp.int32)
    proj = _project_table(emb_w, lin_w, lin_b.reshape(OUT_DIM, 1))
    return _sc_pool(proj, s32)


# speedup vs baseline: 1.2307x; 1.0117x over previous
import functools

import jax
import jax.numpy as jnp
from jax import lax
from jax.experimental import pallas as pl
from jax.experimental.pallas import tpu as pltpu
from jax.experimental.pallas import tpu_sc as plsc

VOCAB = 100000
EMBED_DIM = 128
BATCH = 16384
HIST_LEN = 50
OUT_DIM = 2

PADW = 16
VBLK = 16384
VGRID = -(-VOCAB // VBLK)

NC, NS = 2, 16
NW = NC * NS
BPW = BATCH // NW
CHUNK = 64
GRPC = CHUNK // 16


def _rtne_bf16_hi(u):
    return (u + 0x7FFF + ((u >> 16) & 1)) >> 16


def _proj_body(emb_ref, w_ref, b_ref, out_ref):
    pT = (
        lax.dot_general(
            w_ref[...],
            emb_ref[...],
            (((1,), (1,)), ((), ())),
            preferred_element_type=jnp.float32,
        )
        + b_ref[...]
    )
    u = lax.bitcast_convert_type(pT, jnp.uint32)
    r = _rtne_bf16_hi(u)
    word = r[0:1, :] | (r[1:2, :] << 16)
    out_ref[...] = lax.bitcast_convert_type(word[0], jnp.int32)


def _project_table(emb_w, w_pad, b_pad):
    return pl.pallas_call(
        _proj_body,
        grid=(VGRID,),
        in_specs=[
            pl.BlockSpec((VBLK, EMBED_DIM), lambda i: (i, 0)),
            pl.BlockSpec((OUT_DIM, EMBED_DIM), lambda i: (0, 0)),
            pl.BlockSpec((OUT_DIM, 1), lambda i: (0, 0)),
        ],
        out_specs=pl.BlockSpec((VBLK,), lambda i: (i,)),
        out_shape=jax.ShapeDtypeStruct((VOCAB,), jnp.int32),
    )(emb_w, w_pad, b_pad)


@functools.partial(
    pl.kernel,
    out_type=jax.ShapeDtypeStruct((BATCH, OUT_DIM), jnp.float32),
    mesh=plsc.VectorSubcoreMesh(core_axis_name="c", subcore_axis_name="s"),
    scratch_types=[
        pltpu.VMEM((VOCAB,), jnp.int32),
        pltpu.VMEM((CHUNK, HIST_LEN), jnp.int32),
        pltpu.VMEM((CHUNK, HIST_LEN), jnp.int32),
        pltpu.VMEM((CHUNK, OUT_DIM), jnp.float32),
        pltpu.SemaphoreType.DMA,
        pltpu.SemaphoreType.DMA,
        pltpu.SemaphoreType.DMA,
    ],
    compiler_params=pltpu.CompilerParams(
        use_tc_tiling_on_sc=True, needs_layout_passes=False
    ),
)
def _sc_pool(proj_hbm, s_hbm, out_hbm, table_v, s_va, s_vb, out_v, sem_t, sem_a, sem_b):
    wid = lax.axis_index("s") * NC + lax.axis_index("c")
    base = wid * BPW
    nchunk = BPW // CHUNK
    bufs = (s_va, s_vb)
    sems = (sem_a, sem_b)
    cp_t = pltpu.async_copy(proj_hbm, table_v, sem_t)
    pending = [
        pltpu.async_copy(s_hbm.at[pl.ds(base, CHUNK), :], s_va, sem_a),
        None,
    ]
    cp_t.wait()

    lane = lax.iota(jnp.int32, 16)
    zeros_i = jnp.zeros((16,), jnp.int32)
    zero = jnp.zeros((16,), jnp.float32)
    mask_hi = jnp.full((16,), jnp.int32(-65536))

    for c in range(nchunk):
        if c + 1 < nchunk:
            pending[(c + 1) % 2] = pltpu.async_copy(
                s_hbm.at[pl.ds(base + (c + 1) * CHUNK, CHUNK), :],
                bufs[(c + 1) % 2],
                sems[(c + 1) % 2],
            )
        pending[c % 2].wait()
        s_v = bufs[c % 2]

        def g_body(g, carry):
            rows = [g * 32 + gg * 16 + lane for gg in range(2)]
            accs = [[zero, zero] for _ in range(2)]
            for l in range(HIST_LEN):
                for gg in range(2):
                    v = plsc.load_gather(
                        s_v, [rows[gg], jnp.full((16,), l, jnp.int32)]
                    )
                    w = plsc.load_gather(table_v, [v])
                    accs[gg][0] = accs[gg][0] + plsc.bitcast(w << 16, jnp.float32)
                    accs[gg][1] = accs[gg][1] + plsc.bitcast(
                        w & mask_hi, jnp.float32
                    )
            for gg in range(2):
                plsc.store_scatter(out_v, [rows[gg], zeros_i], accs[gg][0])
                plsc.store_scatter(out_v, [rows[gg], zeros_i + 1], accs[gg][1])
            return carry

        lax.fori_loop(0, GRPC // 2, g_body, 0)
        pltpu.sync_copy(out_v, out_hbm.at[pl.ds(base + c * CHUNK, CHUNK), :])


def kernel(s, emb_w, lin_w, lin_b):
    s32 = s.astype(jnp.int32)
    proj = _project_table(emb_w, lin_w, lin_b.reshape(OUT_DIM, 1))
    return _sc_pool(proj, s32)
